# Initial kernel scaffold; baseline (speedup 1.0000x reference)
import jax
import jax.numpy as jnp
from jax.experimental import pallas as pl


def kernel(input, edge_index, edge_values, W, b):
    raise NotImplementedError("write your pallas kernel here")


# speedup vs baseline: 5.4604x; 5.4604x over previous
import functools

import jax
import jax.numpy as jnp
from jax import lax
from jax.experimental import pallas as pl
from jax.experimental.pallas import tpu as pltpu
from jax.experimental.pallas import tpu_sc as plsc

N_NODES = 10000
N_EDGES = 320000
FEAT = 128
LANES = 16

CHUNK = 128
NCHUNKS = N_EDGES // CHUNK
NC = 2
NS = 16
NW = NC * NS
ROWS_PER_TILE = 624
TAIL_ROWS = N_NODES - NS * ROWS_PER_TILE
TAIL_BASE = NS * ROWS_PER_TILE


def _make_sc_aggregate():
    mesh = plsc.VectorSubcoreMesh(core_axis_name="c", subcore_axis_name="s")

    @functools.partial(
        pl.kernel,
        mesh=mesh,
        out_type=jax.ShapeDtypeStruct((NC * N_NODES, FEAT), jnp.float32),
        scratch_types=[
            pltpu.VMEM((CHUNK,), jnp.int32),
            pltpu.VMEM((CHUNK,), jnp.int32),
            pltpu.VMEM((CHUNK,), jnp.float32),
            pltpu.VMEM((CHUNK, FEAT), jnp.float32),
            pltpu.VMEM_SHARED((N_NODES, FEAT), jnp.float32),
            pltpu.SemaphoreType.DMA,
        ],
    )
    def agg(x_hbm, col_hbm, row_hbm, val_hbm, zero_hbm, out_hbm,
            colv, rowv, valv, xbuf, acc, sem):
        c = lax.axis_index("c")
        s = lax.axis_index("s")
        wid = s * NC + c

        pltpu.sync_copy(zero_hbm,
                        acc.at[pl.ds(s * ROWS_PER_TILE, ROWS_PER_TILE)])

        @pl.when(s == 0)
        def _zero_tail():
            pltpu.sync_copy(zero_hbm.at[pl.ds(0, TAIL_ROWS)],
                            acc.at[pl.ds(TAIL_BASE, TAIL_ROWS)])

        plsc.subcore_barrier()

        nch = (NCHUNKS - wid + NW - 1) // NW

        def body(k, carry):
            ci = wid + k * NW
            base = ci * CHUNK
            pltpu.sync_copy(col_hbm.at[pl.ds(base, CHUNK)], colv)
            pltpu.sync_copy(row_hbm.at[pl.ds(base, CHUNK)], rowv)
            pltpu.sync_copy(val_hbm.at[pl.ds(base, CHUNK)], valv)
            pltpu.async_copy(x_hbm.at[colv], xbuf, sem).wait()
            for g in range(CHUNK // LANES):
                vals16 = valv[pl.ds(g * LANES, LANES)]
                for e in range(LANES):
                    bval = lax.gather(
                        vals16, jnp.full((LANES, 1), e, jnp.int32),
                        lax.GatherDimensionNumbers(
                            offset_dims=(), collapsed_slice_dims=(0,),
                            start_index_map=(0,)),
                        slice_sizes=(1,),
                        mode=lax.GatherScatterMode.PROMISE_IN_BOUNDS)
                    row = g * LANES + e
                    for f in range(FEAT // LANES):
                        sl = pl.ds(f * LANES, LANES)
                        xbuf[row, sl] = xbuf[row, sl] * bval
            pltpu.sync_copy(xbuf, acc.at[rowv], add=True)
            return carry

        lax.fori_loop(0, nch, body, 0)

        plsc.subcore_barrier()
        pltpu.sync_copy(acc.at[pl.ds(s * ROWS_PER_TILE, ROWS_PER_TILE)],
                        out_hbm.at[pl.ds(c * N_NODES + s * ROWS_PER_TILE,
                                         ROWS_PER_TILE)])

        @pl.when(s == 0)
        def _write_tail():
            pltpu.sync_copy(acc.at[pl.ds(TAIL_BASE, TAIL_ROWS)],
                            out_hbm.at[pl.ds(c * N_NODES + TAIL_BASE,
                                             TAIL_ROWS)])

    return agg


_SC_AGG = _make_sc_aggregate()

_BLK = 1000


def _tc_combine(partials, W, b2):
    def body(p0_ref, p1_ref, w_ref, b_ref, o_ref):
        acc = p0_ref[...] + p1_ref[...]
        o_ref[...] = (jnp.dot(acc, w_ref[...],
                              preferred_element_type=jnp.float32)
                      + b_ref[...])

    return pl.pallas_call(
        body,
        grid=(N_NODES // _BLK,),
        in_specs=[
            pl.BlockSpec((_BLK, FEAT), lambda i: (i, 0)),
            pl.BlockSpec((_BLK, FEAT), lambda i: (N_NODES // _BLK + i, 0)),
            pl.BlockSpec((FEAT, FEAT), lambda i: (0, 0)),
            pl.BlockSpec((1, FEAT), lambda i: (0, 0)),
        ],
        out_specs=pl.BlockSpec((_BLK, FEAT), lambda i: (i, 0)),
        out_shape=jax.ShapeDtypeStruct((N_NODES, FEAT), jnp.float32),
    )(partials, partials, W, b2)


def kernel(input, edge_index, edge_values, W, b):
    rows = edge_index[0].astype(jnp.int32)
    cols = edge_index[1].astype(jnp.int32)
    vals = edge_values.astype(jnp.float32)
    zeros = jnp.zeros((ROWS_PER_TILE, FEAT), jnp.float32)
    partials = _SC_AGG(input, cols, rows, vals, zeros)
    return _tc_combine(partials, W, b.reshape(1, FEAT))
